# baseline (speedup 1.0000x reference)
import jax
import jax.numpy as jnp
from jax import lax
from jax.experimental import pallas as pl
from jax.experimental.pallas import tpu as pltpu
from jax.experimental.pallas import tpu_sc as plsc

VOCAB = 1_000_000
HALF = VOCAB // 2
BATCH = 16384
EMBED = 64
NUM_CORES = 2
NUM_SUBCORES = 16
NUM_TILES = NUM_CORES * NUM_SUBCORES
B_PER_TILE = BATCH // NUM_TILES
LANES = 16
N_CHUNKS = B_PER_TILE // LANES


def kernel(inputs, W):
    idx = inputs.reshape((BATCH,))
    w_lo = W[:HALF]
    w_hi = W[HALF:]

    mesh = plsc.VectorSubcoreMesh(core_axis_name="c", subcore_axis_name="s")

    @pl.kernel(
        out_type=jax.ShapeDtypeStruct((BATCH, EMBED), W.dtype),
        mesh=mesh,
        scratch_types=[
            pltpu.VMEM((B_PER_TILE,), jnp.int32),
            pltpu.VMEM((B_PER_TILE,), jnp.int32),
            pltpu.VMEM((B_PER_TILE,), jnp.int32),
            pltpu.VMEM((B_PER_TILE, EMBED), jnp.float32),
            pltpu.VMEM((B_PER_TILE, EMBED), jnp.float32),
            pltpu.SemaphoreType.DMA,
        ],
        compiler_params=pltpu.CompilerParams(
            use_tc_tiling_on_sc=False,
            needs_layout_passes=False,
        ),
    )
    def gather_kernel(lo_hbm, hi_hbm, idx_hbm, out_hbm,
                      idx_v, ilo_v, ihi_v, rows_lo, rows_hi, sem):
        wid = lax.axis_index("s") * NUM_CORES + lax.axis_index("c")
        base = wid * B_PER_TILE
        pltpu.sync_copy(idx_hbm.at[pl.ds(base, B_PER_TILE)], idx_v)

        @pl.loop(0, N_CHUNKS)
        def _(c):
            sl = pl.ds(c * LANES, LANES)
            v = idx_v[sl]
            ge = v >= HALF
            ilo_v[sl] = jnp.where(ge, 0, v)
            ihi_v[sl] = jnp.where(ge, v - HALF, 0)

        a = pltpu.async_copy(lo_hbm.at[ilo_v], rows_lo, sem)
        b = pltpu.async_copy(hi_hbm.at[ihi_v], rows_hi, sem)
        a.wait()
        b.wait()

        lane = lax.broadcasted_iota(jnp.int32, (LANES,), 0)

        @pl.loop(0, N_CHUNKS)
        def _(c):
            chunk = idx_v[pl.ds(c * LANES, LANES)]
            for j in range(LANES):
                s = jnp.sum(jnp.where(lane == j, chunk, 0))
                g = (s >= HALF).astype(jnp.float32)
                r = c * LANES + j
                for l in range(EMBED // LANES):
                    sl = (r, pl.ds(l * LANES, LANES))
                    lo = rows_lo[sl]
                    hi = rows_hi[sl]
                    rows_lo[sl] = lo + (hi - lo) * g
        pltpu.sync_copy(rows_lo, out_hbm.at[pl.ds(base, B_PER_TILE)])

    return gather_kernel(w_lo, w_hi, idx)


# speedup vs baseline: 2.6343x; 2.6343x over previous
import jax
import jax.numpy as jnp
from jax import lax
from jax.experimental import pallas as pl
from jax.experimental.pallas import tpu as pltpu
from jax.experimental.pallas import tpu_sc as plsc

BATCH = 16384
EMBED = 64
NUM_CORES = 2
NUM_SUBCORES = 16
NUM_TILES = NUM_CORES * NUM_SUBCORES
B_PER_TILE = BATCH // NUM_TILES
LANES = 16
N_CHUNKS = B_PER_TILE // LANES
SPLIT = 192
STAGED = B_PER_TILE - SPLIT


def kernel(inputs, W):
    idx = inputs.reshape((BATCH,))

    mesh = plsc.VectorSubcoreMesh(core_axis_name="c", subcore_axis_name="s")

    @pl.kernel(
        out_type=jax.ShapeDtypeStruct((BATCH, EMBED), W.dtype),
        mesh=mesh,
        scratch_types=[
            pltpu.VMEM((B_PER_TILE,), jnp.int32),
            pltpu.VMEM((STAGED, EMBED), jnp.float32),
            pltpu.SemaphoreType.DMA,
            pltpu.SemaphoreType.DMA,
            pltpu.SemaphoreType.DMA,
        ],
        compiler_params=pltpu.CompilerParams(needs_layout_passes=False),
    )
    def gather_kernel(table_hbm, idx_hbm, out_hbm, idx_v, rows_v,
                      sem_i, sem_d, sem_s):
        wid = lax.axis_index("s") * NUM_CORES + lax.axis_index("c")
        base = wid * B_PER_TILE
        pltpu.async_copy(idx_hbm.at[pl.ds(base, B_PER_TILE)], idx_v, sem_i).wait()

        lane = lax.broadcasted_iota(jnp.int32, (LANES,), 0)

        @pl.loop(0, N_CHUNKS)
        def _(c):
            chunk = idx_v[pl.ds(c * LANES, LANES)]
            for j in range(LANES):
                i = jnp.sum(jnp.where(lane == j, chunk, 0))
                b = c * LANES + j

                @pl.when(b < SPLIT)
                def _():
                    pltpu.make_async_copy(
                        table_hbm.at[pl.ds(i, 1)],
                        out_hbm.at[pl.ds(base + b, 1)],
                        sem_d,
                    ).start()

                @pl.when(b >= SPLIT)
                def _():
                    pltpu.make_async_copy(
                        table_hbm.at[pl.ds(i, 1)],
                        rows_v.at[pl.ds(b - SPLIT, 1)],
                        sem_s,
                    ).start()

        @pl.loop(0, SPLIT)
        def _(b):
            pltpu.make_async_copy(
                table_hbm.at[pl.ds(0, 1)],
                out_hbm.at[pl.ds(base + b, 1)],
                sem_d,
            ).wait()

        @pl.loop(0, STAGED)
        def _(b):
            pltpu.make_async_copy(
                table_hbm.at[pl.ds(0, 1)],
                rows_v.at[pl.ds(b, 1)],
                sem_s,
            ).wait()

        pltpu.sync_copy(rows_v, out_hbm.at[pl.ds(base + SPLIT, STAGED)])

    return gather_kernel(W, idx)


# speedup vs baseline: 2.6870x; 1.0200x over previous
import jax
import jax.numpy as jnp
from jax import lax
from jax.experimental import pallas as pl
from jax.experimental.pallas import tpu as pltpu
from jax.experimental.pallas import tpu_sc as plsc

BATCH = 16384
EMBED = 64
NUM_CORES = 2
NUM_SUBCORES = 16
NUM_TILES = NUM_CORES * NUM_SUBCORES
TC_ROWS = 4096
SC_ROWS = BATCH - TC_ROWS
B_PER_TILE = SC_ROWS // NUM_TILES
LANES = 16
N_CHUNKS = B_PER_TILE // LANES


def _tc_gather_kernel(idx_ref, w_ref, o_ref, sem):
    @pl.loop(0, TC_ROWS)
    def _(b):
        i = idx_ref[b]
        pltpu.make_async_copy(
            w_ref.at[pl.ds(i, 1)], o_ref.at[pl.ds(b, 1)], sem
        ).start()

    @pl.loop(0, TC_ROWS)
    def _(b):
        pltpu.make_async_copy(
            w_ref.at[pl.ds(0, 1)], o_ref.at[pl.ds(b, 1)], sem
        ).wait()


def kernel(inputs, W):
    idx = inputs.reshape((BATCH,))
    idx_tc = idx[:TC_ROWS]
    idx_sc = idx[TC_ROWS:]

    out_tc = pl.pallas_call(
        _tc_gather_kernel,
        in_specs=[
            pl.BlockSpec(memory_space=pltpu.SMEM),
            pl.BlockSpec(memory_space=pltpu.HBM),
        ],
        out_specs=pl.BlockSpec(memory_space=pltpu.HBM),
        out_shape=jax.ShapeDtypeStruct((TC_ROWS, EMBED), W.dtype),
        scratch_shapes=[pltpu.SemaphoreType.DMA],
    )(idx_tc, W)

    mesh = plsc.VectorSubcoreMesh(core_axis_name="c", subcore_axis_name="s")

    @pl.kernel(
        out_type=jax.ShapeDtypeStruct((SC_ROWS, EMBED), W.dtype),
        mesh=mesh,
        scratch_types=[
            pltpu.VMEM((B_PER_TILE,), jnp.int32),
            pltpu.VMEM((B_PER_TILE, EMBED), jnp.float32),
            pltpu.SemaphoreType.DMA,
            pltpu.SemaphoreType.DMA,
        ],
        compiler_params=pltpu.CompilerParams(needs_layout_passes=False),
    )
    def sc_gather_kernel(table_hbm, idx_hbm, out_hbm, idx_v, rows_v,
                         sem_i, sem):
        wid = lax.axis_index("s") * NUM_CORES + lax.axis_index("c")
        base = wid * B_PER_TILE
        pltpu.async_copy(idx_hbm.at[pl.ds(base, B_PER_TILE)], idx_v, sem_i).wait()

        lane = lax.broadcasted_iota(jnp.int32, (LANES,), 0)

        @pl.loop(0, N_CHUNKS)
        def _(c):
            chunk = idx_v[pl.ds(c * LANES, LANES)]
            for j in range(LANES):
                i = jnp.sum(jnp.where(lane == j, chunk, 0))
                pltpu.make_async_copy(
                    table_hbm.at[pl.ds(i, 1)],
                    rows_v.at[pl.ds(c * LANES + j, 1)],
                    sem,
                ).start()

        @pl.loop(0, B_PER_TILE)
        def _(b):
            pltpu.make_async_copy(
                table_hbm.at[pl.ds(0, 1)],
                rows_v.at[pl.ds(b, 1)],
                sem,
            ).wait()

        pltpu.sync_copy(rows_v, out_hbm.at[pl.ds(base, B_PER_TILE)])

    out_sc = sc_gather_kernel(W, idx_sc)
    return jnp.concatenate([out_tc, out_sc], axis=0)


# speedup vs baseline: 2.7321x; 1.0168x over previous
import jax
import jax.numpy as jnp
from jax import lax
from jax.experimental import pallas as pl
from jax.experimental.pallas import tpu as pltpu
from jax.experimental.pallas import tpu_sc as plsc

BATCH = 16384
EMBED = 64
NUM_CORES = 2
NUM_SUBCORES = 16
NUM_TILES = NUM_CORES * NUM_SUBCORES
TC_ROWS = 4608
TC_HALF = TC_ROWS // 2
SC_ROWS = BATCH - TC_ROWS
B_PER_TILE = SC_ROWS // NUM_TILES
LANES = 16
N_CHUNKS = B_PER_TILE // LANES
UNROLL = 8


def _tc_gather_kernel(idx_ref, w_ref, o_ref, sem):
    g = pl.program_id(0)
    base = g * TC_HALF

    @pl.loop(0, TC_HALF, step=UNROLL)
    def _(b0):
        for u in range(UNROLL):
            b = base + b0 + u
            i = idx_ref[b]
            pltpu.make_async_copy(
                w_ref.at[pl.ds(i, 1)], o_ref.at[pl.ds(b, 1)], sem
            ).start()

    @pl.loop(0, TC_HALF, step=UNROLL)
    def _(b0):
        for u in range(UNROLL):
            b = base + b0 + u
            pltpu.make_async_copy(
                w_ref.at[pl.ds(0, 1)], o_ref.at[pl.ds(b, 1)], sem
            ).wait()


def kernel(inputs, W):
    idx = inputs.reshape((BATCH,))
    idx_tc = idx[:TC_ROWS]
    idx_sc = idx[TC_ROWS:]

    out_tc = pl.pallas_call(
        _tc_gather_kernel,
        grid=(2,),
        in_specs=[
            pl.BlockSpec(memory_space=pltpu.SMEM),
            pl.BlockSpec(memory_space=pltpu.HBM),
        ],
        out_specs=pl.BlockSpec(memory_space=pltpu.HBM),
        out_shape=jax.ShapeDtypeStruct((TC_ROWS, EMBED), W.dtype),
        scratch_shapes=[pltpu.SemaphoreType.DMA],
        compiler_params=pltpu.CompilerParams(
            dimension_semantics=("parallel",),
        ),
    )(idx_tc, W)

    mesh = plsc.VectorSubcoreMesh(core_axis_name="c", subcore_axis_name="s")

    @pl.kernel(
        out_type=jax.ShapeDtypeStruct((SC_ROWS, EMBED), W.dtype),
        mesh=mesh,
        scratch_types=[
            pltpu.VMEM((B_PER_TILE,), jnp.int32),
            pltpu.VMEM((B_PER_TILE, EMBED), jnp.float32),
            pltpu.SemaphoreType.DMA,
            pltpu.SemaphoreType.DMA,
        ],
        compiler_params=pltpu.CompilerParams(needs_layout_passes=False),
    )
    def sc_gather_kernel(table_hbm, idx_hbm, out_hbm, idx_v, rows_v,
                         sem_i, sem):
        wid = lax.axis_index("s") * NUM_CORES + lax.axis_index("c")
        base = wid * B_PER_TILE
        pltpu.async_copy(idx_hbm.at[pl.ds(base, B_PER_TILE)], idx_v, sem_i).wait()

        lane = lax.broadcasted_iota(jnp.int32, (LANES,), 0)

        @pl.loop(0, N_CHUNKS)
        def _(c):
            chunk = idx_v[pl.ds(c * LANES, LANES)]
            for j in range(LANES):
                i = jnp.sum(jnp.where(lane == j, chunk, 0))
                pltpu.make_async_copy(
                    table_hbm.at[pl.ds(i, 1)],
                    rows_v.at[pl.ds(c * LANES + j, 1)],
                    sem,
                ).start()

        @pl.loop(0, B_PER_TILE)
        def _(b):
            pltpu.make_async_copy(
                table_hbm.at[pl.ds(0, 1)],
                rows_v.at[pl.ds(b, 1)],
                sem,
            ).wait()

        pltpu.sync_copy(rows_v, out_hbm.at[pl.ds(base, B_PER_TILE)])

    out_sc = sc_gather_kernel(W, idx_sc)
    return jnp.concatenate([out_tc, out_sc], axis=0)


# speedup vs baseline: 2.8855x; 1.0561x over previous
import jax
import jax.numpy as jnp
from jax import lax
from jax.experimental import pallas as pl
from jax.experimental.pallas import tpu as pltpu
from jax.experimental.pallas import tpu_sc as plsc

BATCH = 16384
EMBED = 64
NUM_CORES = 2
NUM_SUBCORES = 16
NUM_TILES = NUM_CORES * NUM_SUBCORES
TC_ROWS = 3072
TC_HALF = TC_ROWS // 2
SC_ROWS = BATCH - TC_ROWS
B_PER_TILE = SC_ROWS // NUM_TILES
LANES = 16
N_CHUNKS = B_PER_TILE // LANES
UNROLL = 8


def _tc_gather_kernel(idx_ref, w_ref, o_ref, sem):
    g = pl.program_id(0)
    base = g * TC_HALF

    @pl.loop(0, TC_HALF, step=UNROLL)
    def _(b0):
        for u in range(UNROLL):
            b = base + b0 + u
            i = idx_ref[b]
            pltpu.make_async_copy(
                w_ref.at[pl.ds(i, 1)], o_ref.at[pl.ds(b, 1)], sem
            ).start()

    @pl.loop(0, TC_HALF, step=UNROLL)
    def _(b0):
        for u in range(UNROLL):
            b = base + b0 + u
            pltpu.make_async_copy(
                w_ref.at[pl.ds(0, 1)], o_ref.at[pl.ds(b, 1)], sem
            ).wait()


def kernel(inputs, W):
    idx = inputs.reshape((BATCH,))
    idx_tc = idx[:TC_ROWS]
    idx_sc = idx[TC_ROWS:]

    out_tc = pl.pallas_call(
        _tc_gather_kernel,
        grid=(2,),
        in_specs=[
            pl.BlockSpec(memory_space=pltpu.SMEM),
            pl.BlockSpec(memory_space=pltpu.HBM),
        ],
        out_specs=pl.BlockSpec(memory_space=pltpu.HBM),
        out_shape=jax.ShapeDtypeStruct((TC_ROWS, EMBED), W.dtype),
        scratch_shapes=[pltpu.SemaphoreType.DMA],
        compiler_params=pltpu.CompilerParams(
            dimension_semantics=("parallel",),
        ),
    )(idx_tc, W)

    mesh = plsc.VectorSubcoreMesh(core_axis_name="c", subcore_axis_name="s")

    @pl.kernel(
        out_type=jax.ShapeDtypeStruct((SC_ROWS, EMBED), W.dtype),
        mesh=mesh,
        scratch_types=[
            pltpu.VMEM((B_PER_TILE,), jnp.int32),
            pltpu.VMEM((B_PER_TILE, EMBED), jnp.float32),
            pltpu.SemaphoreType.DMA,
            pltpu.SemaphoreType.DMA,
        ],
        compiler_params=pltpu.CompilerParams(needs_layout_passes=False),
    )
    def sc_gather_kernel(table_hbm, idx_hbm, out_hbm, idx_v, rows_v,
                         sem_i, sem):
        wid = lax.axis_index("s") * NUM_CORES + lax.axis_index("c")
        base = wid * B_PER_TILE
        pltpu.async_copy(idx_hbm.at[pl.ds(base, B_PER_TILE)], idx_v, sem_i).wait()

        lane = lax.broadcasted_iota(jnp.int32, (LANES,), 0)

        @pl.loop(0, N_CHUNKS)
        def _(c):
            chunk = idx_v[pl.ds(c * LANES, LANES)]
            for j in range(LANES):
                i = jnp.sum(jnp.where(lane == j, chunk, 0))
                pltpu.make_async_copy(
                    table_hbm.at[pl.ds(i, 1)],
                    rows_v.at[pl.ds(c * LANES + j, 1)],
                    sem,
                ).start()

        @pl.loop(0, B_PER_TILE)
        def _(b):
            pltpu.make_async_copy(
                table_hbm.at[pl.ds(0, 1)],
                rows_v.at[pl.ds(b, 1)],
                sem,
            ).wait()

        pltpu.sync_copy(rows_v, out_hbm.at[pl.ds(base, B_PER_TILE)])

    out_sc = sc_gather_kernel(W, idx_sc)
    return jnp.concatenate([out_tc, out_sc], axis=0)


# speedup vs baseline: 3.2714x; 1.1337x over previous
import jax
import jax.numpy as jnp
from jax import lax
from jax.experimental import pallas as pl
from jax.experimental.pallas import tpu as pltpu
from jax.experimental.pallas import tpu_sc as plsc

BATCH = 16384
EMBED = 64
NUM_CORES = 2
NUM_SUBCORES = 16
NUM_TILES = NUM_CORES * NUM_SUBCORES
B_PER_TILE = BATCH // NUM_TILES
LANES = 16
N_CHUNKS = B_PER_TILE // LANES


def kernel(inputs, W):
    idx = inputs.reshape((BATCH,))

    mesh = plsc.VectorSubcoreMesh(core_axis_name="c", subcore_axis_name="s")

    @pl.kernel(
        out_type=jax.ShapeDtypeStruct((BATCH, EMBED), W.dtype),
        mesh=mesh,
        scratch_types=[
            pltpu.VMEM((B_PER_TILE,), jnp.int32),
            pltpu.VMEM((B_PER_TILE, EMBED), jnp.float32),
            pltpu.SemaphoreType.DMA,
            pltpu.SemaphoreType.DMA,
        ],
        compiler_params=pltpu.CompilerParams(needs_layout_passes=False),
    )
    def gather_kernel(table_hbm, idx_hbm, out_hbm, idx_v, rows_v, sem_i, sem):
        wid = lax.axis_index("s") * NUM_CORES + lax.axis_index("c")
        base = wid * B_PER_TILE
        pltpu.async_copy(idx_hbm.at[pl.ds(base, B_PER_TILE)], idx_v, sem_i).wait()

        lane = lax.broadcasted_iota(jnp.int32, (LANES,), 0)

        @pl.loop(0, N_CHUNKS)
        def _(c):
            chunk = idx_v[pl.ds(c * LANES, LANES)]
            for j in range(LANES):
                i = jnp.sum(jnp.where(lane == j, chunk, 0))
                pltpu.make_async_copy(
                    table_hbm.at[pl.ds(i, 1)],
                    rows_v.at[pl.ds(c * LANES + j, 1)],
                    sem,
                ).start()

        @pl.loop(0, B_PER_TILE)
        def _(b):
            pltpu.make_async_copy(
                table_hbm.at[pl.ds(0, 1)],
                rows_v.at[pl.ds(b, 1)],
                sem,
            ).wait()

        pltpu.sync_copy(rows_v, out_hbm.at[pl.ds(base, B_PER_TILE)])

    return gather_kernel(W, idx)
